# baseline (speedup 1.0000x reference)
import jax
import jax.numpy as jnp
from jax.experimental import pallas as pl

_K = 64


def _matvec_body(x_ref, w_ref, o_ref):
    x = x_ref[0]
    w = w_ref[...]
    s = jax.lax.dot_general(x, w, (((1,), (0,)), ((), ())),
                            preferred_element_type=jnp.float32)
    o_ref[0] = jnp.maximum(s, 0.0)


def _threshold_body(s_ref, o_ref):
    s = s_ref[...]
    bits = jax.lax.bitcast_convert_type(s, jnp.int32)
    B = s.shape[0]

    def step(_, lh):
        lo, hi = lh
        mid = lo + ((hi - lo + 1) >> 1)
        cnt = jnp.sum((bits >= mid).astype(jnp.int32), axis=1, keepdims=True)
        ge = cnt >= _K
        return jnp.where(ge, mid, lo), jnp.where(ge, hi, mid - 1)

    lo0 = jnp.zeros((B, 1), jnp.int32)
    hi0 = jnp.full((B, 1), 0x7F7FFFFF, jnp.int32)
    lo, _ = jax.lax.fori_loop(0, 31, step, (lo0, hi0))

    m = (bits >= lo).astype(jnp.float32)
    num = jnp.exp(s) * m
    den = jnp.sum(num, axis=1, keepdims=True)
    o_ref[...] = num / den


def kernel(x0, wa):
    B, N, D = x0.shape
    R = 32
    KDIM = D * R
    M = N // R
    x6 = x0.reshape(B, M, KDIM)
    w32 = jnp.kron(jnp.eye(R, dtype=x0.dtype), wa)

    scores = pl.pallas_call(
        _matvec_body,
        grid=(B,),
        in_specs=[pl.BlockSpec((1, M, KDIM), lambda i: (i, 0, 0)),
                  pl.BlockSpec((KDIM, R), lambda i: (0, 0))],
        out_specs=pl.BlockSpec((1, M, R), lambda i: (i, 0, 0)),
        out_shape=jax.ShapeDtypeStruct((B, M, R), jnp.float32),
    )(x6, w32)

    s2 = scores.reshape(B, N)
    out = pl.pallas_call(
        _threshold_body,
        in_specs=[pl.BlockSpec((B, N), lambda: (0, 0))],
        out_specs=pl.BlockSpec((B, N), lambda: (0, 0)),
        out_shape=jax.ShapeDtypeStruct((B, N), jnp.float32),
    )(s2)
    return out.reshape(B, N, 1)


# speedup vs baseline: 6.6026x; 6.6026x over previous
import jax
import jax.numpy as jnp
from jax.experimental import pallas as pl

_K = 64
_BB = 4


def _matvec_body(x_ref, w_ref, o_ref):
    bb, d, n = x_ref.shape
    x = x_ref[...].reshape(bb * d, n)
    w = w_ref[...]
    s = jax.lax.dot_general(w, x, (((1,), (0,)), ((), ())),
                            preferred_element_type=jnp.float32)
    o_ref[0] = jnp.maximum(s, 0.0)


def _threshold_body(s_ref, o_ref):
    s = s_ref[...]
    bits = jax.lax.bitcast_convert_type(s, jnp.int32)
    B = s.shape[0]

    def step(_, lh):
        lo, hi = lh
        mid = lo + ((hi - lo + 1) >> 1)
        cnt = jnp.sum((bits >= mid).astype(jnp.int32), axis=1, keepdims=True)
        ge = cnt >= _K
        return jnp.where(ge, mid, lo), jnp.where(ge, hi, mid - 1)

    lo0 = jnp.zeros((B, 1), jnp.int32)
    hi0 = jnp.full((B, 1), 0x7F7FFFFF, jnp.int32)
    lo, _ = jax.lax.fori_loop(0, 31, step, (lo0, hi0))

    m = (bits >= lo).astype(jnp.float32)
    num = jnp.exp(s) * m
    den = jnp.sum(num, axis=1, keepdims=True)
    o_ref[...] = num / den


def kernel(x0, wa):
    B, N, D = x0.shape
    xt = jnp.transpose(x0, (0, 2, 1))
    wrow = wa.reshape(1, D)
    wbd = jnp.kron(jnp.eye(_BB, dtype=x0.dtype), wrow)

    scores = pl.pallas_call(
        _matvec_body,
        grid=(B // _BB,),
        in_specs=[pl.BlockSpec((_BB, D, N), lambda i: (i, 0, 0)),
                  pl.BlockSpec((_BB, _BB * D), lambda i: (0, 0))],
        out_specs=pl.BlockSpec((1, _BB, N), lambda i: (i, 0, 0)),
        out_shape=jax.ShapeDtypeStruct((B // _BB, _BB, N), jnp.float32),
    )(xt, wbd)
    scores = scores.reshape(B, N)

    out = pl.pallas_call(
        _threshold_body,
        in_specs=[pl.BlockSpec((B, N), lambda: (0, 0))],
        out_specs=pl.BlockSpec((B, N), lambda: (0, 0)),
        out_shape=jax.ShapeDtypeStruct((B, N), jnp.float32),
    )(scores)
    return out.reshape(B, N, 1)
